# baseline (speedup 1.0000x reference)
import functools

import jax
import jax.numpy as jnp
from jax import lax
from jax.experimental import pallas as pl
from jax.experimental.pallas import tpu as pltpu
from jax.experimental.pallas import tpu_sc as plsc

VOCAB = 100000
EMBED_DIM = 300
BATCH = 1024
MAX_NORM = 1.0
V_TILE = 2048


@functools.lru_cache(maxsize=None)
def _sc_gather():
    info = plsc.get_sparse_core_info()
    nw = info.num_cores * info.num_subcores
    b_per_w = BATCH // nw
    mesh = plsc.VectorSubcoreMesh(core_axis_name="c", subcore_axis_name="s")

    @functools.partial(
        pl.kernel,
        mesh=mesh,
        out_type=jax.ShapeDtypeStruct((BATCH, EMBED_DIM), jnp.float32),
        scratch_types=[
            pltpu.VMEM((b_per_w,), jnp.int32),
            pltpu.VMEM((b_per_w, EMBED_DIM), jnp.float32),
            pltpu.SemaphoreType.DMA,
        ],
        compiler_params=pltpu.CompilerParams(needs_layout_passes=False),
    )
    def gather(table_hbm, idx_hbm, out_hbm, idx_v, rows_v, sem):
        wid = lax.axis_index("s") * info.num_cores + lax.axis_index("c")
        base = wid * b_per_w
        pltpu.sync_copy(idx_hbm.at[pl.ds(base, b_per_w)], idx_v)
        lanes = lax.iota(jnp.int32, 16)
        copies = []
        for i in range(b_per_w):
            vec = idx_v[pl.ds((i // 16) * 16, 16)]
            r = jnp.sum(jnp.where(lanes == (i % 16), vec, 0))
            c = pltpu.make_async_copy(
                table_hbm.at[pl.ds(r, 1)],
                rows_v.at[pl.ds(i, 1)],
                sem,
            )
            c.start()
            copies.append(c)
        for c in copies:
            c.wait()
        pltpu.sync_copy(rows_v, out_hbm.at[pl.ds(base, b_per_w)])

    return gather


def _proj_body(emb_ref, w_ref, b_ref, out_ref, esc_ref):
    @pl.when(pl.program_id(0) == 0)
    def _():
        e = emb_ref[...]
        nrm = jnp.sqrt(jnp.sum(e * e, axis=1, keepdims=True))
        scale = jnp.where(nrm > MAX_NORM, MAX_NORM / jnp.maximum(nrm, 1e-12), 1.0)
        esc_ref[...] = (e * scale).astype(jnp.bfloat16)

    out_ref[...] = lax.dot_general(
        esc_ref[...],
        w_ref[...].astype(jnp.bfloat16),
        dimension_numbers=(((1,), (1,)), ((), ())),
        preferred_element_type=jnp.float32,
    ) + b_ref[...]


def _tc_project(emb_raw, W, b2, interpret=False):
    grid = pl.cdiv(VOCAB, V_TILE)
    return pl.pallas_call(
        _proj_body,
        grid=(grid,),
        in_specs=[
            pl.BlockSpec((BATCH, EMBED_DIM), lambda j: (0, 0)),
            pl.BlockSpec((V_TILE, EMBED_DIM), lambda j: (j, 0)),
            pl.BlockSpec((1, V_TILE), lambda j: (0, j)),
        ],
        out_specs=pl.BlockSpec((BATCH, V_TILE), lambda j: (0, j)),
        out_shape=jax.ShapeDtypeStruct((BATCH, VOCAB), jnp.float32),
        scratch_shapes=[pltpu.VMEM((BATCH, EMBED_DIM), jnp.bfloat16)],
        compiler_params=pltpu.CompilerParams(
            dimension_semantics=("arbitrary",)
        ),
        interpret=interpret,
    )(emb_raw, W, b2)


def kernel(inputs, emb_table, W, b):
    idx = inputs.reshape(-1).astype(jnp.int32)
    emb_raw = _sc_gather()(emb_table, idx)
    return _tc_project(emb_raw, W, b.reshape(1, VOCAB))


# speedup vs baseline: 1.0065x; 1.0065x over previous
import functools

import jax
import jax.numpy as jnp
from jax import lax
from jax.experimental import pallas as pl
from jax.experimental.pallas import tpu as pltpu
from jax.experimental.pallas import tpu_sc as plsc

VOCAB = 100000
EMBED_DIM = 300
BATCH = 1024
MAX_NORM = 1.0
V_TILE = 4096


@functools.lru_cache(maxsize=None)
def _sc_gather():
    info = plsc.get_sparse_core_info()
    nw = info.num_cores * info.num_subcores
    b_per_w = BATCH // nw
    mesh = plsc.VectorSubcoreMesh(core_axis_name="c", subcore_axis_name="s")

    @functools.partial(
        pl.kernel,
        mesh=mesh,
        out_type=jax.ShapeDtypeStruct((BATCH, EMBED_DIM), jnp.float32),
        scratch_types=[
            pltpu.VMEM((b_per_w,), jnp.int32),
            pltpu.VMEM((b_per_w, EMBED_DIM), jnp.float32),
            pltpu.SemaphoreType.DMA,
        ],
        compiler_params=pltpu.CompilerParams(needs_layout_passes=False),
    )
    def gather(table_hbm, idx_hbm, out_hbm, idx_v, rows_v, sem):
        wid = lax.axis_index("s") * info.num_cores + lax.axis_index("c")
        base = wid * b_per_w
        pltpu.sync_copy(idx_hbm.at[pl.ds(base, b_per_w)], idx_v)
        lanes = lax.iota(jnp.int32, 16)
        copies = []
        for i in range(b_per_w):
            vec = idx_v[pl.ds((i // 16) * 16, 16)]
            r = jnp.sum(jnp.where(lanes == (i % 16), vec, 0))
            c = pltpu.make_async_copy(
                table_hbm.at[pl.ds(r, 1)],
                rows_v.at[pl.ds(i, 1)],
                sem,
            )
            c.start()
            copies.append(c)
        for c in copies:
            c.wait()
        pltpu.sync_copy(rows_v, out_hbm.at[pl.ds(base, b_per_w)])

    return gather


def _proj_body(emb_ref, w_ref, b_ref, out_ref, esc_ref):
    @pl.when(pl.program_id(0) == 0)
    def _():
        e = emb_ref[...]
        nrm = jnp.sqrt(jnp.sum(e * e, axis=1, keepdims=True))
        scale = jnp.where(nrm > MAX_NORM, MAX_NORM / jnp.maximum(nrm, 1e-12), 1.0)
        esc_ref[...] = (e * scale).astype(jnp.bfloat16)

    out_ref[...] = lax.dot_general(
        esc_ref[...],
        w_ref[...].astype(jnp.bfloat16),
        dimension_numbers=(((1,), (1,)), ((), ())),
        preferred_element_type=jnp.float32,
    ) + b_ref[...]


def _tc_project(emb_raw, W, b2, interpret=False):
    grid = pl.cdiv(VOCAB, V_TILE)
    return pl.pallas_call(
        _proj_body,
        grid=(grid,),
        in_specs=[
            pl.BlockSpec((BATCH, EMBED_DIM), lambda j: (0, 0)),
            pl.BlockSpec((V_TILE, EMBED_DIM), lambda j: (j, 0)),
            pl.BlockSpec((1, V_TILE), lambda j: (0, j)),
        ],
        out_specs=pl.BlockSpec((BATCH, V_TILE), lambda j: (0, j)),
        out_shape=jax.ShapeDtypeStruct((BATCH, VOCAB), jnp.float32),
        scratch_shapes=[pltpu.VMEM((BATCH, EMBED_DIM), jnp.bfloat16)],
        compiler_params=pltpu.CompilerParams(
            dimension_semantics=("arbitrary",)
        ),
        interpret=interpret,
    )(emb_raw, W, b2)


def kernel(inputs, emb_table, W, b):
    idx = inputs.reshape(-1).astype(jnp.int32)
    emb_raw = _sc_gather()(emb_table, idx)
    return _tc_project(emb_raw, W, b.reshape(1, VOCAB))


# speedup vs baseline: 1.0070x; 1.0005x over previous
import functools

import jax
import jax.numpy as jnp
from jax import lax
from jax.experimental import pallas as pl
from jax.experimental.pallas import tpu as pltpu
from jax.experimental.pallas import tpu_sc as plsc

VOCAB = 100000
EMBED_DIM = 300
BATCH = 1024
MAX_NORM = 1.0
V_TILE = 4096


@functools.lru_cache(maxsize=None)
def _sc_gather():
    info = plsc.get_sparse_core_info()
    nw = info.num_cores * info.num_subcores
    b_per_w = BATCH // nw
    mesh = plsc.VectorSubcoreMesh(core_axis_name="c", subcore_axis_name="s")

    @functools.partial(
        pl.kernel,
        mesh=mesh,
        out_type=jax.ShapeDtypeStruct((BATCH, EMBED_DIM), jnp.float32),
        scratch_types=[
            pltpu.VMEM((b_per_w,), jnp.int32),
            pltpu.VMEM((b_per_w, EMBED_DIM), jnp.float32),
            pltpu.SemaphoreType.DMA,
        ],
        compiler_params=pltpu.CompilerParams(needs_layout_passes=False),
    )
    def gather(table_hbm, idx_hbm, out_hbm, idx_v, rows_v, sem):
        wid = lax.axis_index("s") * info.num_cores + lax.axis_index("c")
        base = wid * b_per_w
        pltpu.sync_copy(idx_hbm.at[pl.ds(base, b_per_w)], idx_v)
        lanes = lax.iota(jnp.int32, 16)
        copies = []
        for i in range(b_per_w):
            vec = idx_v[pl.ds((i // 16) * 16, 16)]
            r = jnp.sum(jnp.where(lanes == (i % 16), vec, 0))
            c = pltpu.make_async_copy(
                table_hbm.at[pl.ds(r, 1)],
                rows_v.at[pl.ds(i, 1)],
                sem,
            )
            c.start()
            copies.append(c)
        for c in copies:
            c.wait()
        pltpu.sync_copy(rows_v, out_hbm.at[pl.ds(base, b_per_w)])

    return gather


def _proj_body(emb_ref, w_ref, b_ref, out_ref):
    e = emb_ref[...]
    nrm = jnp.sqrt(jnp.sum(e * e, axis=1, keepdims=True))
    scale = jnp.where(nrm > MAX_NORM, MAX_NORM / jnp.maximum(nrm, 1e-12), 1.0)
    esc = (e * scale).astype(jnp.bfloat16)

    out_ref[...] = lax.dot_general(
        esc,
        w_ref[...].astype(jnp.bfloat16),
        dimension_numbers=(((1,), (1,)), ((), ())),
        preferred_element_type=jnp.float32,
    ) + b_ref[...]


def _tc_project(emb_raw, W, b2, interpret=False):
    grid = pl.cdiv(VOCAB, V_TILE)
    return pl.pallas_call(
        _proj_body,
        grid=(grid,),
        in_specs=[
            pl.BlockSpec((BATCH, EMBED_DIM), lambda j: (0, 0)),
            pl.BlockSpec((V_TILE, EMBED_DIM), lambda j: (j, 0)),
            pl.BlockSpec((1, V_TILE), lambda j: (0, j)),
        ],
        out_specs=pl.BlockSpec((BATCH, V_TILE), lambda j: (0, j)),
        out_shape=jax.ShapeDtypeStruct((BATCH, VOCAB), jnp.float32),
        compiler_params=pltpu.CompilerParams(
            dimension_semantics=("parallel",)
        ),
        interpret=interpret,
    )(emb_raw, W, b2)


def kernel(inputs, emb_table, W, b):
    idx = inputs.reshape(-1).astype(jnp.int32)
    emb_raw = _sc_gather()(emb_table, idx)
    return _tc_project(emb_raw, W, b.reshape(1, VOCAB))


# speedup vs baseline: 2.4900x; 2.4727x over previous
import functools

import jax
import jax.numpy as jnp
from jax import lax
from jax.experimental import pallas as pl
from jax.experimental.pallas import tpu as pltpu
from jax.experimental.pallas import tpu_sc as plsc

VOCAB = 100000
EMBED_DIM = 300
BATCH = 1024
MAX_NORM = 1.0
V_TILE = 2048


@functools.lru_cache(maxsize=None)
def _sc_gather():
    info = plsc.get_sparse_core_info()
    nw = info.num_cores * info.num_subcores
    b_per_w = BATCH // nw
    mesh = plsc.VectorSubcoreMesh(core_axis_name="c", subcore_axis_name="s")

    @functools.partial(
        pl.kernel,
        mesh=mesh,
        out_type=jax.ShapeDtypeStruct((BATCH, EMBED_DIM), jnp.float32),
        scratch_types=[
            pltpu.VMEM((b_per_w,), jnp.int32),
            pltpu.VMEM((b_per_w, EMBED_DIM), jnp.float32),
            pltpu.SemaphoreType.DMA,
        ],
        compiler_params=pltpu.CompilerParams(needs_layout_passes=False),
    )
    def gather(table_hbm, idx_hbm, out_hbm, idx_v, rows_v, sem):
        wid = lax.axis_index("s") * info.num_cores + lax.axis_index("c")
        base = wid * b_per_w
        pltpu.sync_copy(idx_hbm.at[pl.ds(base, b_per_w)], idx_v)
        lanes = lax.iota(jnp.int32, 16)
        copies = []
        for i in range(b_per_w):
            vec = idx_v[pl.ds((i // 16) * 16, 16)]
            r = jnp.sum(jnp.where(lanes == (i % 16), vec, 0))
            c = pltpu.make_async_copy(
                table_hbm.at[pl.ds(r, 1)],
                rows_v.at[pl.ds(i, 1)],
                sem,
            )
            c.start()
            copies.append(c)
        for c in copies:
            c.wait()
        pltpu.sync_copy(rows_v, out_hbm.at[pl.ds(base, b_per_w)])

    return gather


def _proj_body(emb_t_ref, w_t_ref, b_ref, out_t_ref):
    e = emb_t_ref[...]
    nrm = jnp.sqrt(jnp.sum(e * e, axis=0, keepdims=True))
    scale = jnp.where(nrm > MAX_NORM, MAX_NORM / jnp.maximum(nrm, 1e-12), 1.0)
    esc = (e * scale).astype(jnp.bfloat16)

    acc = lax.dot_general(
        w_t_ref[...].astype(jnp.bfloat16),
        esc,
        dimension_numbers=(((0,), (0,)), ((), ())),
        preferred_element_type=jnp.float32,
    )
    out_t_ref[...] = acc + jnp.transpose(b_ref[...])


def _tc_project(emb_t, W_t, b2, interpret=False):
    grid = pl.cdiv(VOCAB, V_TILE)
    return pl.pallas_call(
        _proj_body,
        grid=(grid,),
        in_specs=[
            pl.BlockSpec((EMBED_DIM, BATCH), lambda j: (0, 0)),
            pl.BlockSpec((EMBED_DIM, V_TILE), lambda j: (0, j)),
            pl.BlockSpec((1, V_TILE), lambda j: (0, j)),
        ],
        out_specs=pl.BlockSpec((V_TILE, BATCH), lambda j: (j, 0)),
        out_shape=jax.ShapeDtypeStruct((VOCAB, BATCH), jnp.float32),
        compiler_params=pltpu.CompilerParams(
            dimension_semantics=("arbitrary",)
        ),
        interpret=interpret,
    )(emb_t, W_t, b2)


def kernel(inputs, emb_table, W, b):
    idx = inputs.reshape(-1).astype(jnp.int32)
    emb = _sc_gather()(emb_table, idx)
    out_t = _tc_project(emb.T, W.T, b.reshape(1, VOCAB))
    return out_t.T


# speedup vs baseline: 2.9018x; 1.1654x over previous
import functools

import jax
import jax.numpy as jnp
from jax import lax
from jax.experimental import pallas as pl
from jax.experimental.pallas import tpu as pltpu
from jax.experimental.pallas import tpu_sc as plsc

VOCAB = 100000
EMBED_DIM = 300
BATCH = 1024
MAX_NORM = 1.0
V_TILE = 2048


DPAD = 304


@functools.lru_cache(maxsize=None)
def _sc_gather():
    info = plsc.get_sparse_core_info()
    nw = info.num_cores * info.num_subcores
    b_per_w = BATCH // nw
    n_chunks = DPAD // 16
    mesh = plsc.VectorSubcoreMesh(core_axis_name="c", subcore_axis_name="s")

    @functools.partial(
        pl.kernel,
        mesh=mesh,
        out_type=jax.ShapeDtypeStruct((BATCH, DPAD), jnp.float32),
        scratch_types=[
            pltpu.VMEM((b_per_w,), jnp.int32),
            pltpu.VMEM((EMBED_DIM, 128), jnp.float32),
            pltpu.VMEM((EMBED_DIM, 128), jnp.float32),
            pltpu.VMEM((b_per_w, DPAD), jnp.float32),
            pltpu.SemaphoreType.DMA,
            pltpu.SemaphoreType.DMA,
        ],
        compiler_params=pltpu.CompilerParams(needs_layout_passes=False),
    )
    def gather(table_t_hbm, idx_hbm, out_hbm, idx_v, blk0, blk1, rows_v, sem0, sem1):
        wid = lax.axis_index("s") * info.num_cores + lax.axis_index("c")
        base = wid * b_per_w
        pltpu.sync_copy(idx_hbm.at[pl.ds(base, b_per_w)], idx_v)
        lanes = lax.iota(jnp.int32, 16)
        blks = (blk0, blk1)
        sems = (sem0, sem1)

        def col_of(i):
            vec = idx_v[pl.ds((i // 16) * 16, 16)]
            return jnp.sum(jnp.where(lanes == (i % 16), vec, 0))

        def start_fetch(i):
            r = col_of(i)
            g = pl.multiple_of((r // 128) * 128, 128)
            c = pltpu.make_async_copy(
                table_t_hbm.at[:, pl.ds(g, 128)], blks[i % 2], sems[i % 2]
            )
            c.start()
            return c

        pend = start_fetch(0)
        for i in range(b_per_w):
            pend.wait()
            nxt = start_fetch(i + 1) if i + 1 < b_per_w else None
            r = col_of(i)
            lvec = jnp.broadcast_to(r % 128, (16,))
            blk = blks[i % 2]
            for j in range(n_chunks):
                ks = j * 16 + lanes
                vals = plsc.load_gather(blk, [jnp.minimum(ks, EMBED_DIM - 1), lvec])
                vals = jnp.where(ks < EMBED_DIM, vals, 0.0)
                rows_v[i, pl.ds(j * 16, 16)] = vals
            pend = nxt
        pltpu.sync_copy(rows_v, out_hbm.at[pl.ds(base, b_per_w)])

    return gather


def _proj_body(emb_t_ref, w_t_ref, b_ref, out_t_ref):
    e = emb_t_ref[...]
    nrm = jnp.sqrt(jnp.sum(e * e, axis=0, keepdims=True))
    scale = jnp.where(nrm > MAX_NORM, MAX_NORM / jnp.maximum(nrm, 1e-12), 1.0)
    esc = (e * scale).astype(jnp.bfloat16)

    acc = lax.dot_general(
        w_t_ref[...].astype(jnp.bfloat16),
        esc,
        dimension_numbers=(((0,), (0,)), ((), ())),
        preferred_element_type=jnp.float32,
    )
    out_t_ref[...] = acc + jnp.transpose(b_ref[...])


def _tc_project(emb_t, W_t, b2, interpret=False):
    grid = pl.cdiv(VOCAB, V_TILE)
    return pl.pallas_call(
        _proj_body,
        grid=(grid,),
        in_specs=[
            pl.BlockSpec((EMBED_DIM, BATCH), lambda j: (0, 0)),
            pl.BlockSpec((EMBED_DIM, V_TILE), lambda j: (0, j)),
            pl.BlockSpec((1, V_TILE), lambda j: (0, j)),
        ],
        out_specs=pl.BlockSpec((V_TILE, BATCH), lambda j: (j, 0)),
        out_shape=jax.ShapeDtypeStruct((VOCAB, BATCH), jnp.float32),
        compiler_params=pltpu.CompilerParams(
            dimension_semantics=("arbitrary",)
        ),
        interpret=interpret,
    )(emb_t, W_t, b2)


def kernel(inputs, emb_table, W, b):
    idx = inputs.reshape(-1).astype(jnp.int32)
    emb = _sc_gather()(emb_table.T, idx)[:, :EMBED_DIM]
    out_t = _tc_project(emb.T, W.T, b.reshape(1, VOCAB))
    return out_t.T


# speedup vs baseline: 2.9711x; 1.0239x over previous
import functools

import jax
import jax.numpy as jnp
from jax import lax
from jax.experimental import pallas as pl
from jax.experimental.pallas import tpu as pltpu
from jax.experimental.pallas import tpu_sc as plsc

VOCAB = 100000
EMBED_DIM = 300
BATCH = 1024
MAX_NORM = 1.0
V_TILE = 4096


DPAD = 304


@functools.lru_cache(maxsize=None)
def _sc_gather():
    info = plsc.get_sparse_core_info()
    nw = info.num_cores * info.num_subcores
    b_per_w = BATCH // nw
    n_chunks = DPAD // 16
    mesh = plsc.VectorSubcoreMesh(core_axis_name="c", subcore_axis_name="s")

    @functools.partial(
        pl.kernel,
        mesh=mesh,
        out_type=jax.ShapeDtypeStruct((BATCH, DPAD), jnp.float32),
        scratch_types=[
            pltpu.VMEM((b_per_w,), jnp.int32),
            pltpu.VMEM((EMBED_DIM, 128), jnp.float32),
            pltpu.VMEM((EMBED_DIM, 128), jnp.float32),
            pltpu.VMEM((b_per_w, DPAD), jnp.float32),
            pltpu.SemaphoreType.DMA,
            pltpu.SemaphoreType.DMA,
        ],
        compiler_params=pltpu.CompilerParams(needs_layout_passes=False),
    )
    def gather(table_t_hbm, idx_hbm, out_hbm, idx_v, blk0, blk1, rows_v, sem0, sem1):
        wid = lax.axis_index("s") * info.num_cores + lax.axis_index("c")
        base = wid * b_per_w
        pltpu.sync_copy(idx_hbm.at[pl.ds(base, b_per_w)], idx_v)
        lanes = lax.iota(jnp.int32, 16)
        blks = (blk0, blk1)
        sems = (sem0, sem1)

        def col_of(i):
            vec = idx_v[pl.ds((i // 16) * 16, 16)]
            return jnp.sum(jnp.where(lanes == (i % 16), vec, 0))

        def start_fetch(i):
            r = col_of(i)
            g = pl.multiple_of((r // 128) * 128, 128)
            c = pltpu.make_async_copy(
                table_t_hbm.at[:, pl.ds(g, 128)], blks[i % 2], sems[i % 2]
            )
            c.start()
            return c

        pend = start_fetch(0)
        for i in range(b_per_w):
            pend.wait()
            nxt = start_fetch(i + 1) if i + 1 < b_per_w else None
            r = col_of(i)
            lvec = jnp.broadcast_to(r % 128, (16,))
            blk = blks[i % 2]
            for j in range(n_chunks):
                ks = j * 16 + lanes
                vals = plsc.load_gather(blk, [jnp.minimum(ks, EMBED_DIM - 1), lvec])
                vals = jnp.where(ks < EMBED_DIM, vals, 0.0)
                rows_v[i, pl.ds(j * 16, 16)] = vals
            pend = nxt
        pltpu.sync_copy(rows_v, out_hbm.at[pl.ds(base, b_per_w)])

    return gather


def _proj_body(emb_t_ref, w_t_ref, b_ref, out_t_ref):
    e = emb_t_ref[...]
    nrm = jnp.sqrt(jnp.sum(e * e, axis=0, keepdims=True))
    scale = jnp.where(nrm > MAX_NORM, MAX_NORM / jnp.maximum(nrm, 1e-12), 1.0)
    esc = (e * scale).astype(jnp.bfloat16)

    acc = lax.dot_general(
        w_t_ref[...].astype(jnp.bfloat16),
        esc,
        dimension_numbers=(((0,), (0,)), ((), ())),
        preferred_element_type=jnp.float32,
    )
    out_t_ref[...] = acc + jnp.transpose(b_ref[...])


def _tc_project(emb_t, W_t, b2, interpret=False):
    grid = pl.cdiv(VOCAB, V_TILE)
    return pl.pallas_call(
        _proj_body,
        grid=(grid,),
        in_specs=[
            pl.BlockSpec((EMBED_DIM, BATCH), lambda j: (0, 0)),
            pl.BlockSpec((EMBED_DIM, V_TILE), lambda j: (0, j)),
            pl.BlockSpec((1, V_TILE), lambda j: (0, j)),
        ],
        out_specs=pl.BlockSpec((V_TILE, BATCH), lambda j: (j, 0)),
        out_shape=jax.ShapeDtypeStruct((VOCAB, BATCH), jnp.float32),
        compiler_params=pltpu.CompilerParams(
            dimension_semantics=("arbitrary",)
        ),
        interpret=interpret,
    )(emb_t, W_t, b2)


def kernel(inputs, emb_table, W, b):
    idx = inputs.reshape(-1).astype(jnp.int32)
    emb = _sc_gather()(emb_table.T, idx)[:, :EMBED_DIM]
    out_t = _tc_project(emb.T, W.T, b.reshape(1, VOCAB))
    return out_t.T
